# baseline (speedup 1.0000x reference)
import functools

import jax
import jax.numpy as jnp
from jax import lax
from jax.experimental import pallas as pl
from jax.experimental.pallas import tpu as pltpu
from jax.experimental.pallas import tpu_sc as plsc

_B = 4096
_D = 128
_NB = 2
_NW = 32
_CR = 128
_MBLK = 24
_BR = 8192
_PBLK = 16


def _sc_body(te_in, ti_in, tip_in, idx_in, z_in, step_in,
             te_out, ti_out, tip_out,
             buf0, buf1, ibuf, svec, rsem, wsem):
    wid = lax.axis_index("s") * 2 + lax.axis_index("c")

    pltpu.sync_copy(step_in, svec)
    s = jnp.max(svec[...])
    ps = lax.rem(s, jnp.int32(16))

    bufs = (buf0, buf1)
    row0 = wid * _CR

    def rd_start(c):
        @pl.when(c != s)
        def _():
            pltpu.make_async_copy(
                te_in.at[pl.ds(c * _B + row0, _CR), :], bufs[c % 2], rsem
            ).start()

        @pl.when(c == s)
        def _():
            pltpu.make_async_copy(
                z_in.at[pl.ds(row0, _CR), :], bufs[c % 2], rsem).start()

    def rd_wait(c):
        pltpu.make_async_copy(
            z_in.at[pl.ds(row0, _CR), :], bufs[c % 2], rsem).wait()

    def wr(c):
        return pltpu.make_async_copy(
            bufs[c % 2], te_out.at[pl.ds(c * _B + row0, _CR), :], wsem)

    rd_start(0)
    for c in range(_MBLK):
        if c >= 1:
            wr(c - 1).wait()
        if c + 1 < _MBLK:
            rd_start(c + 1)
        rd_wait(c)
        wr(c).start()
    wr(_MBLK - 1).wait()

    @pl.when(wid < _MBLK)
    def _():
        @pl.when(wid != s)
        def _():
            pltpu.sync_copy(ti_in.at[pl.ds(wid * _B, _B)], ibuf)

        @pl.when(wid == s)
        def _():
            pltpu.sync_copy(idx_in, ibuf)

        pltpu.sync_copy(ibuf, ti_out.at[pl.ds(wid * _B, _B)])

    @pl.when(wid < 16)
    def _():
        @pl.when(wid != ps)
        def _():
            pltpu.sync_copy(tip_in.at[pl.ds(wid * _B, _B)], ibuf)

        @pl.when(wid == ps)
        def _():
            pltpu.sync_copy(idx_in, ibuf)

        pltpu.sync_copy(ibuf, tip_out.at[pl.ds(wid * _B, _B)])


def _tc_body(step_ref, tep_in, emb_in, tep_out):
    i = pl.program_id(0)
    s = step_ref[0]
    ps = lax.rem(s, jnp.int32(16))
    j = lax.rem(i, _PBLK // _NB)
    pblk, poff = ps // 2, lax.rem(ps, 2) * _B

    tep_out[...] = tep_in[...]

    @pl.when(j == pblk)
    def _():
        tep_out[pl.ds(poff, _B), :] = emb_in[0]


def _tc_update_pos(train_embeddings_pos, embeddings, step):
    P = train_embeddings_pos.shape[1]
    tep_flat = train_embeddings_pos.reshape(_NB * P, _D)

    out = pl.pallas_call(
        _tc_body,
        grid_spec=pltpu.PrefetchScalarGridSpec(
            num_scalar_prefetch=1,
            grid=(_PBLK,),
            in_specs=[
                pl.BlockSpec((_BR, _D), lambda i, s: (i, 0)),
                pl.BlockSpec((1, _B, _D),
                             lambda i, s: (i // (_PBLK // _NB), 0, 0)),
            ],
            out_specs=pl.BlockSpec((_BR, _D), lambda i, s: (i, 0)),
        ),
        out_shape=jax.ShapeDtypeStruct((_NB * P, _D), jnp.float32),
        compiler_params=pltpu.CompilerParams(
            dimension_semantics=("arbitrary",),
        ),
    )(step, tep_flat, embeddings)
    return out.reshape(_NB, P, _D)


def kernel(train_indices_ref, train_embeddings_ref, train_indices_pos,
           train_embeddings_pos, indices, Z_ssps, embeddings, step_rel):
    M = train_embeddings_ref.shape[0]
    P = train_indices_pos.shape[0]
    step = jnp.asarray(step_rel, jnp.int32)
    step_vec = jnp.full((16,), step, jnp.int32)

    sc = functools.partial(
        pl.kernel,
        out_type=[
            jax.ShapeDtypeStruct((M, _D), jnp.float32),
            jax.ShapeDtypeStruct((M,), jnp.int32),
            jax.ShapeDtypeStruct((P,), jnp.int32),
        ],
        mesh=plsc.VectorSubcoreMesh(core_axis_name="c", subcore_axis_name="s"),
        scratch_types=[
            pltpu.VMEM((_CR, _D), jnp.float32),
            pltpu.VMEM((_CR, _D), jnp.float32),
            pltpu.VMEM((_B,), jnp.int32),
            pltpu.VMEM((16,), jnp.int32),
            pltpu.SemaphoreType.DMA,
            pltpu.SemaphoreType.DMA,
        ],
        compiler_params=pltpu.CompilerParams(needs_layout_passes=False),
    )(_sc_body)

    te_out, ti_out, tip_out = sc(
        train_embeddings_ref, train_indices_ref, train_indices_pos, indices,
        Z_ssps, step_vec)

    tep_out = _tc_update_pos(train_embeddings_pos, embeddings,
                             step.reshape(1))

    return (ti_out, te_out, tip_out, tep_out)
